# Initial kernel scaffold; baseline (speedup 1.0000x reference)
import jax
import jax.numpy as jnp
from jax.experimental import pallas as pl


def kernel(x, adj, adj_w, W):
    raise NotImplementedError("write your pallas kernel here")


# speedup vs baseline: 1.0075x; 1.0075x over previous
import jax
import jax.numpy as jnp
from jax.experimental import pallas as pl

N = 10000
D = 128
ROWS = 200


def _support_kernel(x_ref, w_ref, o_ref):
    o_ref[...] = jax.lax.dot(
        x_ref[...], w_ref[...], preferred_element_type=jnp.float32
    )


def _gcn_kernel(adj_ref, adjw_ref, s_ref, o_ref):
    a = adj_ref[...] + adjw_ref[...]
    out = jax.lax.dot(a, s_ref[...], preferred_element_type=jnp.float32)
    norm = jnp.sqrt(jnp.sum(out * out, axis=-1, keepdims=True))
    o_ref[...] = out / jnp.maximum(norm, 1e-12)


def kernel(x, adj, adj_w, W):
    support = pl.pallas_call(
        _support_kernel,
        out_shape=jax.ShapeDtypeStruct((N, D), jnp.float32),
    )(x, W)

    return pl.pallas_call(
        _gcn_kernel,
        grid=(N // ROWS,),
        in_specs=[
            pl.BlockSpec((ROWS, N), lambda i: (i, 0)),
            pl.BlockSpec((ROWS, N), lambda i: (i, 0)),
            pl.BlockSpec((N, D), lambda i: (0, 0)),
        ],
        out_specs=pl.BlockSpec((ROWS, D), lambda i: (i, 0)),
        out_shape=jax.ShapeDtypeStruct((N, D), jnp.float32),
    )(adj, adj_w, support)


# speedup vs baseline: 1.0280x; 1.0203x over previous
import jax
import jax.numpy as jnp
from jax.experimental import pallas as pl
from jax.experimental.pallas import tpu as pltpu

N = 10000
D = 128
ROWS = 200


def _gcn_kernel(x_ref, w_ref, adj_ref, adjw_ref, o_ref, s_ref):
    @pl.when(pl.program_id(0) == 0)
    def _():
        s_ref[...] = jax.lax.dot(
            x_ref[...], w_ref[...], preferred_element_type=jnp.float32
        )

    a = adj_ref[...] + adjw_ref[...]
    out = jax.lax.dot(a, s_ref[...], preferred_element_type=jnp.float32)
    norm = jnp.sqrt(jnp.sum(out * out, axis=-1, keepdims=True))
    o_ref[...] = out / jnp.maximum(norm, 1e-12)


def kernel(x, adj, adj_w, W):
    return pl.pallas_call(
        _gcn_kernel,
        grid=(N // ROWS,),
        in_specs=[
            pl.BlockSpec((N, D), lambda i: (0, 0)),
            pl.BlockSpec((D, D), lambda i: (0, 0)),
            pl.BlockSpec((ROWS, N), lambda i: (i, 0)),
            pl.BlockSpec((ROWS, N), lambda i: (i, 0)),
        ],
        out_specs=pl.BlockSpec((ROWS, D), lambda i: (i, 0)),
        out_shape=jax.ShapeDtypeStruct((N, D), jnp.float32),
        scratch_shapes=[pltpu.VMEM((N, D), jnp.float32)],
    )(x, W, adj, adj_w)
